# baseline (speedup 1.0000x reference)
import functools

import jax
import jax.numpy as jnp
from jax import lax
from jax.experimental import pallas as pl
from jax.experimental.pallas import tpu as pltpu
from jax.experimental.pallas import tpu_sc as plsc

_NC = 2
_NS = 16
_L = 16
_NW = _NC * _NS

_CHUNK = 12800


def _sc_lookup(conv16, species_flat):
    n = species_flat.shape[0]
    per_w = n // _NW
    n_chunks = per_w // _CHUNK
    mesh = plsc.VectorSubcoreMesh(core_axis_name="c", subcore_axis_name="s")

    @functools.partial(
        pl.kernel,
        out_type=jax.ShapeDtypeStruct((n,), jnp.int32),
        mesh=mesh,
        scratch_types=[
            pltpu.VMEM((_L,), jnp.int32),
            pltpu.VMEM((_CHUNK,), jnp.int32),
        ],
        compiler_params=pltpu.CompilerParams(needs_layout_passes=False),
    )
    def k(conv_hbm, sp_hbm, out_hbm, table_v, buf_v):
        wid = lax.axis_index("s") * _NC + lax.axis_index("c")
        base = wid * per_w
        pltpu.sync_copy(conv_hbm, table_v)

        @pl.loop(0, n_chunks)
        def _chunk(ci):
            off = base + ci * _CHUNK
            pltpu.sync_copy(sp_hbm.at[pl.ds(off, _CHUNK)], buf_v)

            @pl.loop(0, _CHUNK // _L, unroll=8)
            def _vec(vi):
                s = pl.ds(vi * _L, _L)
                buf_v[s] = plsc.load_gather(table_v, [buf_v[s]])

            pltpu.sync_copy(buf_v, out_hbm.at[pl.ds(off, _CHUNK)])

    return k(conv16, species_flat)


def kernel(species, conv_tensor):
    conv16 = jnp.pad(conv_tensor, (0, _L - conv_tensor.shape[0]))
    out = _sc_lookup(conv16, species.reshape(-1))
    return out.reshape(species.shape)


# speedup vs baseline: 1.2487x; 1.2487x over previous
import functools

import jax
import jax.numpy as jnp
from jax import lax
from jax.experimental import pallas as pl
from jax.experimental.pallas import tpu as pltpu
from jax.experimental.pallas import tpu_sc as plsc

_NC = 2
_NS = 16
_L = 16
_NW = _NC * _NS

_CHUNK = 12800


def _sc_lookup(conv16, species_flat):
    n = species_flat.shape[0]
    per_w = n // _NW
    n_chunks = per_w // _CHUNK
    mesh = plsc.VectorSubcoreMesh(core_axis_name="c", subcore_axis_name="s")

    @functools.partial(
        pl.kernel,
        out_type=jax.ShapeDtypeStruct((n,), jnp.int32),
        mesh=mesh,
        scratch_types=[
            pltpu.VMEM((_L,), jnp.int32),
            pltpu.VMEM((_CHUNK,), jnp.int32),
            pltpu.VMEM((_CHUNK,), jnp.int32),
            pltpu.VMEM((_CHUNK,), jnp.int32),
            pltpu.VMEM((_CHUNK,), jnp.int32),
            pltpu.SemaphoreType.DMA,
            pltpu.SemaphoreType.DMA,
            pltpu.SemaphoreType.DMA,
            pltpu.SemaphoreType.DMA,
        ],
        compiler_params=pltpu.CompilerParams(needs_layout_passes=False),
    )
    def k(conv_hbm, sp_hbm, out_hbm, table_v, in0, in1, out0, out1,
          si0, si1, so0, so1):
        ins, outs = (in0, in1), (out0, out1)
        isems, osems = (si0, si1), (so0, so1)
        wid = lax.axis_index("s") * _NC + lax.axis_index("c")
        base = wid * per_w
        pltpu.sync_copy(conv_hbm, table_v)

        for b in range(2):
            pltpu.async_copy(
                sp_hbm.at[pl.ds(base + b * _CHUNK, _CHUNK)], ins[b], isems[b])

        for c in range(n_chunks):
            b = c % 2
            off = base + c * _CHUNK
            pltpu.make_async_copy(
                sp_hbm.at[pl.ds(off, _CHUNK)], ins[b], isems[b]).wait()
            if c >= 2:
                pltpu.make_async_copy(
                    outs[b], out_hbm.at[pl.ds(off, _CHUNK)], osems[b]).wait()

            @plsc.parallel_loop(0, _CHUNK // _L, unroll=8)
            def _vec(vi, _in=ins[b], _out=outs[b]):
                s = pl.ds(vi * _L, _L)
                _out[s] = plsc.load_gather(table_v, [_in[s]])

            pltpu.async_copy(outs[b], out_hbm.at[pl.ds(off, _CHUNK)], osems[b])
            if c + 2 < n_chunks:
                off2 = base + (c + 2) * _CHUNK
                pltpu.async_copy(
                    sp_hbm.at[pl.ds(off2, _CHUNK)], ins[b], isems[b])

        for b in range(min(2, n_chunks)):
            pltpu.make_async_copy(
                outs[b], out_hbm.at[pl.ds(base, _CHUNK)], osems[b]).wait()

    return k(conv16, species_flat)


def kernel(species, conv_tensor):
    conv16 = jnp.pad(conv_tensor, (0, _L - conv_tensor.shape[0]))
    out = _sc_lookup(conv16, species.reshape(-1))
    return out.reshape(species.shape)


# speedup vs baseline: 2.1688x; 1.7369x over previous
import functools

import jax
import jax.numpy as jnp
from jax import lax
from jax.experimental import pallas as pl
from jax.experimental.pallas import tpu as pltpu
from jax.experimental.pallas import tpu_sc as plsc

_NC = 2
_NS = 16
_L = 16
_NW = _NC * _NS

_ROWS = 64


def _sc_lookup(conv_tensor, species):
    nrows, ncols = species.shape
    per_w = nrows // _NW
    n_chunks = per_w // _ROWS
    nvec = ncols // _L
    tails = [] if ncols % _L == 0 else [ncols - _L]
    mesh = plsc.VectorSubcoreMesh(core_axis_name="c", subcore_axis_name="s")

    @functools.partial(
        pl.kernel,
        out_type=jax.ShapeDtypeStruct((nrows, ncols), jnp.int32),
        mesh=mesh,
        scratch_types=[
            pltpu.VMEM((conv_tensor.shape[0],), jnp.int32),
            pltpu.VMEM((_ROWS, ncols), jnp.int32),
            pltpu.VMEM((_ROWS, ncols), jnp.int32),
            pltpu.VMEM((_ROWS, ncols), jnp.int32),
            pltpu.VMEM((_ROWS, ncols), jnp.int32),
            pltpu.SemaphoreType.DMA,
            pltpu.SemaphoreType.DMA,
            pltpu.SemaphoreType.DMA,
            pltpu.SemaphoreType.DMA,
        ],
        compiler_params=pltpu.CompilerParams(needs_layout_passes=False),
    )
    def k(conv_hbm, sp_hbm, out_hbm, table_v, in0, in1, out0, out1,
          si0, si1, so0, so1):
        ins, outs = (in0, in1), (out0, out1)
        isems, osems = (si0, si1), (so0, so1)
        wid = lax.axis_index("s") * _NC + lax.axis_index("c")
        base = wid * per_w
        pltpu.sync_copy(conv_hbm, table_v)

        for b in range(2):
            pltpu.async_copy(
                sp_hbm.at[pl.ds(base + b * _ROWS, _ROWS)], ins[b], isems[b])

        for c in range(n_chunks):
            b = c % 2
            off = base + c * _ROWS
            pltpu.make_async_copy(
                sp_hbm.at[pl.ds(off, _ROWS)], ins[b], isems[b]).wait()
            if c >= 2:
                pltpu.make_async_copy(
                    outs[b], out_hbm.at[pl.ds(off, _ROWS)], osems[b]).wait()

            @plsc.parallel_loop(0, _ROWS, unroll=2)
            def _row(r, _in=ins[b], _out=outs[b]):
                for col in [j * _L for j in range(nvec)] + tails:
                    s = pl.ds(col, _L)
                    _out[r, s] = plsc.load_gather(table_v, [_in[r, s]])

            pltpu.async_copy(outs[b], out_hbm.at[pl.ds(off, _ROWS)], osems[b])
            if c + 2 < n_chunks:
                off2 = base + (c + 2) * _ROWS
                pltpu.async_copy(
                    sp_hbm.at[pl.ds(off2, _ROWS)], ins[b], isems[b])

        for b in range(min(2, n_chunks)):
            pltpu.make_async_copy(
                outs[b], out_hbm.at[pl.ds(base, _ROWS)], osems[b]).wait()

    return k(conv_tensor, species)


def kernel(species, conv_tensor):
    return _sc_lookup(conv_tensor, species)


# speedup vs baseline: 2.1715x; 1.0012x over previous
import functools

import jax
import jax.numpy as jnp
from jax import lax
from jax.experimental import pallas as pl
from jax.experimental.pallas import tpu as pltpu
from jax.experimental.pallas import tpu_sc as plsc

_NC = 2
_NS = 16
_L = 16
_NW = _NC * _NS

_ROWS = 64


def _sc_lookup(conv_tensor, species):
    nrows, ncols = species.shape
    per_w = nrows // _NW
    n_chunks = per_w // _ROWS
    nvec = ncols // _L
    tails = [] if ncols % _L == 0 else [ncols - _L]
    mesh = plsc.VectorSubcoreMesh(core_axis_name="c", subcore_axis_name="s")

    @functools.partial(
        pl.kernel,
        out_type=jax.ShapeDtypeStruct((nrows, ncols), jnp.int32),
        mesh=mesh,
        scratch_types=[
            pltpu.VMEM((conv_tensor.shape[0],), jnp.int32),
            pltpu.VMEM((_ROWS, ncols), jnp.int32),
            pltpu.VMEM((_ROWS, ncols), jnp.int32),
            pltpu.VMEM((_ROWS, ncols), jnp.int32),
            pltpu.VMEM((_ROWS, ncols), jnp.int32),
            pltpu.SemaphoreType.DMA,
            pltpu.SemaphoreType.DMA,
            pltpu.SemaphoreType.DMA,
            pltpu.SemaphoreType.DMA,
        ],
        compiler_params=pltpu.CompilerParams(
            needs_layout_passes=False, use_tc_tiling_on_sc=True),
    )
    def k(conv_hbm, sp_hbm, out_hbm, table_v, in0, in1, out0, out1,
          si0, si1, so0, so1):
        ins, outs = (in0, in1), (out0, out1)
        isems, osems = (si0, si1), (so0, so1)
        wid = lax.axis_index("s") * _NC + lax.axis_index("c")
        base = wid * per_w
        pltpu.sync_copy(conv_hbm, table_v)

        for b in range(2):
            pltpu.async_copy(
                sp_hbm.at[pl.ds(base + b * _ROWS, _ROWS)], ins[b], isems[b])

        for c in range(n_chunks):
            b = c % 2
            off = base + c * _ROWS
            pltpu.make_async_copy(
                sp_hbm.at[pl.ds(off, _ROWS)], ins[b], isems[b]).wait()
            if c >= 2:
                pltpu.make_async_copy(
                    outs[b], out_hbm.at[pl.ds(off, _ROWS)], osems[b]).wait()

            @plsc.parallel_loop(0, _ROWS, unroll=2)
            def _row(r, _in=ins[b], _out=outs[b]):
                for col in [j * _L for j in range(nvec)] + tails:
                    s = pl.ds(col, _L)
                    _out[r, s] = plsc.load_gather(table_v, [_in[r, s]])

            pltpu.async_copy(outs[b], out_hbm.at[pl.ds(off, _ROWS)], osems[b])
            if c + 2 < n_chunks:
                off2 = base + (c + 2) * _ROWS
                pltpu.async_copy(
                    sp_hbm.at[pl.ds(off2, _ROWS)], ins[b], isems[b])

        for b in range(min(2, n_chunks)):
            pltpu.make_async_copy(
                outs[b], out_hbm.at[pl.ds(base, _ROWS)], osems[b]).wait()

    return k(conv_tensor, species)


def kernel(species, conv_tensor):
    return _sc_lookup(conv_tensor, species)


# speedup vs baseline: 3.3095x; 1.5241x over previous
import functools

import jax
import jax.numpy as jnp
from jax import lax
from jax.experimental import pallas as pl
from jax.experimental.pallas import tpu as pltpu
from jax.experimental.pallas import tpu_sc as plsc

_NC = 2
_NS = 16
_L = 16
_NW = _NC * _NS

_ROWS = 40


def _sc_lookup(conv_tensor, sp_t):
    nrows, ncols = sp_t.shape
    stripe = ncols // _NW
    n_chunks = nrows // _ROWS
    nvec = stripe // _L
    mesh = plsc.VectorSubcoreMesh(core_axis_name="c", subcore_axis_name="s")

    @functools.partial(
        pl.kernel,
        out_type=jax.ShapeDtypeStruct((nrows, ncols), jnp.int32),
        mesh=mesh,
        scratch_types=[
            pltpu.VMEM((conv_tensor.shape[0],), jnp.int32),
            pltpu.VMEM((_ROWS, stripe), jnp.int32),
            pltpu.VMEM((_ROWS, stripe), jnp.int32),
            pltpu.VMEM((_ROWS, stripe), jnp.int32),
            pltpu.VMEM((_ROWS, stripe), jnp.int32),
            pltpu.SemaphoreType.DMA,
            pltpu.SemaphoreType.DMA,
            pltpu.SemaphoreType.DMA,
            pltpu.SemaphoreType.DMA,
        ],
        compiler_params=pltpu.CompilerParams(
            needs_layout_passes=False, use_tc_tiling_on_sc=True),
    )
    def k(conv_hbm, sp_hbm, out_hbm, table_v, in0, in1, out0, out1,
          si0, si1, so0, so1):
        ins, outs = (in0, in1), (out0, out1)
        isems, osems = (si0, si1), (so0, so1)
        wid = lax.axis_index("s") * _NC + lax.axis_index("c")
        col0 = wid * stripe
        pltpu.sync_copy(conv_hbm, table_v)

        def src(c):
            return sp_hbm.at[pl.ds(c * _ROWS, _ROWS), pl.ds(col0, stripe)]

        def dst(c):
            return out_hbm.at[pl.ds(c * _ROWS, _ROWS), pl.ds(col0, stripe)]

        for b in range(2):
            pltpu.async_copy(src(b), ins[b], isems[b])

        for c in range(n_chunks):
            b = c % 2
            pltpu.make_async_copy(src(c), ins[b], isems[b]).wait()
            if c >= 2:
                pltpu.make_async_copy(outs[b], dst(c), osems[b]).wait()

            @plsc.parallel_loop(0, _ROWS, unroll=2)
            def _row(r, _in=ins[b], _out=outs[b]):
                for j in range(nvec):
                    s = pl.ds(j * _L, _L)
                    _out[r, s] = plsc.load_gather(table_v, [_in[r, s]])

            pltpu.async_copy(outs[b], dst(c), osems[b])
            if c + 2 < n_chunks:
                pltpu.async_copy(src(c + 2), ins[b], isems[b])

        for b in range(min(2, n_chunks)):
            pltpu.make_async_copy(outs[b], dst(0), osems[b]).wait()

    return k(conv_tensor, sp_t)


def kernel(species, conv_tensor):
    return _sc_lookup(conv_tensor, species.T).T
